# Initial kernel scaffold; baseline (speedup 1.0000x reference)
import jax
import jax.numpy as jnp
from jax.experimental import pallas as pl


def kernel(x, edge_index, W1_l, b1, W1_r, W2_l, b2, W2_r):
    raise NotImplementedError("write your pallas kernel here")


# speedup vs baseline: 1.2156x; 1.2156x over previous
import jax
import jax.numpy as jnp
from jax import lax
from jax.experimental import pallas as pl
from jax.experimental.pallas import tpu as pltpu
from jax.experimental.pallas import tpu_sc as plsc

N = 10000
E = 160000
D_IN = 256
D_HID = 512
W = 128
NCH1 = D_IN // W
NCH2 = D_HID // W

NC = 2
NS = 16
CHUNK = 128
CPS = 80
E_PAD = NS * CPS * CHUNK

NH = N // 2
N_ACC = 5008
DUMMY = NH
ZROWS = 312
ZLAST = N_ACC - 15 * ZROWS
OROWS = 312
OLAST = NH - 15 * OROWS

_MESH = plsc.VectorSubcoreMesh(core_axis_name="c", subcore_axis_name="s")


def _make_agg(n_chunks: int, with_cnt: bool):
    per_core = n_chunks // NC

    out_type = [jax.ShapeDtypeStruct((N, W), jnp.float32) for _ in range(n_chunks)]
    if with_cnt:
        out_type.append(jax.ShapeDtypeStruct((N, W), jnp.float32))

    n_in = n_chunks + (4 if with_cnt else 3)
    n_out = n_chunks + (1 if with_cnt else 0)

    scratch = [
        pltpu.VMEM((CHUNK,), jnp.int32),
        pltpu.VMEM((CHUNK,), jnp.int32),
        pltpu.VMEM((CHUNK,), jnp.int32),
        pltpu.VMEM((CHUNK, W), jnp.float32),
        pltpu.VMEM_SHARED((N_ACC, W), jnp.float32),
        pltpu.SemaphoreType.DMA,
    ]

    def body(*refs):
        x_refs = refs[:n_chunks]
        src_r, dst_r, z_acc = refs[n_chunks:n_chunks + 3]
        ones_h = refs[n_chunks + 3] if with_cnt else None
        o_refs = refs[n_in:n_in + n_chunks]
        cnt_o = refs[n_in + n_chunks] if with_cnt else None
        src_v, dst_v, scidx_v, rows_v, acc_sh, sem = refs[n_in + n_out:]

        core = lax.axis_index("c")
        sub = lax.axis_index("s")

        def _zero_acc():
            @pl.when(sub < 15)
            def _():
                pltpu.sync_copy(z_acc.at[pl.ds(sub * ZROWS, ZROWS)],
                                acc_sh.at[pl.ds(sub * ZROWS, ZROWS)])

            @pl.when(sub == 15)
            def _():
                pltpu.sync_copy(z_acc.at[pl.ds(15 * ZROWS, ZLAST)],
                                acc_sh.at[pl.ds(15 * ZROWS, ZLAST)])

        def _copy_out(o_ref, half):
            @pl.when(sub < 15)
            def _():
                pltpu.sync_copy(
                    acc_sh.at[pl.ds(sub * OROWS, OROWS)],
                    o_ref.at[pl.ds(half * NH + sub * OROWS, OROWS)])

            @pl.when(sub == 15)
            def _():
                pltpu.sync_copy(
                    acc_sh.at[pl.ds(15 * OROWS, OLAST)],
                    o_ref.at[pl.ds(half * NH + 15 * OROWS, OLAST)])

        def _transform(half):
            base = half * NH
            for g in range(CHUNK // 16):
                d = dst_v[pl.ds(g * 16, 16)]
                dl = d - base
                ok = (dl >= 0) & (dl < NH)
                scidx_v[pl.ds(g * 16, 16)] = jnp.where(ok, dl, DUMMY)

        def run_pass(x_ref, o_ref, half):
            _zero_acc()
            if x_ref is None:
                pltpu.sync_copy(ones_h, rows_v)
            plsc.subcore_barrier()

            def step(j, carry):
                pltpu.sync_copy(dst_r.at[sub, j, 0], dst_v)
                _transform(half)
                if x_ref is not None:
                    pltpu.sync_copy(src_r.at[sub, j, 0], src_v)
                    pltpu.async_copy(x_ref.at[src_v], rows_v, sem).wait()
                pltpu.sync_copy(rows_v, acc_sh.at[scidx_v], add=True)
                return carry

            lax.fori_loop(0, CPS, step, 0)
            plsc.subcore_barrier()
            _copy_out(o_ref, half)
            plsc.subcore_barrier()

        for cid in range(NC):
            @pl.when(core == cid)
            def _(cid=cid):
                for k in range(per_core):
                    c = cid * per_core + k
                    for half in range(2):
                        run_pass(x_refs[c], o_refs[c], half)
                if with_cnt:
                    run_pass(None, cnt_o, cid)

    return pl.kernel(body, out_type=out_type, mesh=_MESH,
                     scratch_types=scratch)


_agg1 = _make_agg(NCH1, with_cnt=True)
_agg2 = _make_agg(NCH2, with_cnt=False)

BN = 400


def _combine1_body(*refs):
    s = refs[:NCH1]
    cnt, x, wl, wr, b = refs[NCH1:NCH1 + 5]
    o = refs[NCH1 + 5:]
    r = 1.0 / jnp.maximum(cnt[:, 0:1], 1.0)
    acc = jnp.dot(x[...], wr[...], preferred_element_type=jnp.float32)
    for c in range(NCH1):
        acc += jnp.dot(s[c][...] * r, wl[c * W:(c + 1) * W, :],
                       preferred_element_type=jnp.float32)
    h = jnp.maximum(acc + b[...], 0.0)
    for c in range(NCH2):
        o[c][...] = h[:, c * W:(c + 1) * W]


def _combine2_body(*refs):
    s = refs[:NCH2]
    cnt = refs[NCH2]
    h = refs[NCH2 + 1:NCH2 + 1 + NCH2]
    wl, wr, b, out = refs[NCH2 + 1 + NCH2:]
    r = 1.0 / jnp.maximum(cnt[:, 0:1], 1.0)
    acc = b[...] + jnp.zeros((BN, D_HID), jnp.float32)
    for c in range(NCH2):
        acc += jnp.dot(s[c][...] * r, wl[c * W:(c + 1) * W, :],
                       preferred_element_type=jnp.float32)
        acc += jnp.dot(h[c][...], wr[c * W:(c + 1) * W, :],
                       preferred_element_type=jnp.float32)
    out[...] = acc


def _row_block(d):
    return pl.BlockSpec((BN, d), lambda i: (i, 0))


def _full(shape):
    return pl.BlockSpec(shape, lambda i: tuple(0 for _ in shape))


_combine1 = pl.pallas_call(
    _combine1_body,
    grid=(N // BN,),
    in_specs=[_row_block(W)] * NCH1 + [_row_block(W), _row_block(D_IN),
              _full((D_IN, D_HID)), _full((D_IN, D_HID)), _full((1, D_HID))],
    out_specs=[_row_block(W)] * NCH2,
    out_shape=[jax.ShapeDtypeStruct((N, W), jnp.float32)] * NCH2,
)

_combine2 = pl.pallas_call(
    _combine2_body,
    grid=(N // BN,),
    in_specs=[_row_block(W)] * NCH2 + [_row_block(W)] + [_row_block(W)] * NCH2
             + [_full((D_HID, D_HID)), _full((D_HID, D_HID)), _full((1, D_HID))],
    out_specs=_row_block(D_HID),
    out_shape=jax.ShapeDtypeStruct((N, D_HID), jnp.float32),
)


def kernel(x, edge_index, W1_l, b1, W1_r, W2_l, b2, W2_r):
    src = edge_index[0].astype(jnp.int32)
    dst = edge_index[1].astype(jnp.int32)
    pad = E_PAD - E
    src_p = jnp.concatenate([src, jnp.zeros((pad,), jnp.int32)])
    src_p = src_p.reshape(NS, CPS, 1, CHUNK)
    dst_p = jnp.concatenate([dst, jnp.full((pad,), N, jnp.int32)])
    dst_p = dst_p.reshape(NS, CPS, 1, CHUNK)

    xc = [x[:, c * W:(c + 1) * W] for c in range(NCH1)]
    z_acc = jnp.zeros((N_ACC, W), jnp.float32)
    ones_h = jnp.ones((CHUNK, W), jnp.float32)

    s0, s1, cnt = _agg1(*xc, src_p, dst_p, z_acc, ones_h)
    hc = _combine1(s0, s1, cnt, x, W1_l, W1_r, b1.reshape(1, D_HID))
    t = _agg2(*hc, src_p, dst_p, z_acc)
    out = _combine2(*t, cnt, *hc, W2_l, W2_r, b2.reshape(1, D_HID))
    return out


# speedup vs baseline: 1.4024x; 1.1536x over previous
import jax
import jax.numpy as jnp
from jax import lax
from jax.experimental import pallas as pl
from jax.experimental.pallas import tpu as pltpu
from jax.experimental.pallas import tpu_sc as plsc

N = 10000
E = 160000
D_IN = 256
D_HID = 512
W = 128
NCH1 = D_IN // W
NCH2 = D_HID // W

NC = 2
NS = 16
CHUNK = 64
CPS = 160
HCPS = CPS // 2
E_PAD = NS * CPS * CHUNK

NH = N // 2
N_ACC = 5008
DUMMY = NH
ZROWS = 312
ZLAST = N_ACC - 15 * ZROWS
OROWS = 312
OLAST = NH - 15 * OROWS

_MESH = plsc.VectorSubcoreMesh(core_axis_name="c", subcore_axis_name="s")


def _make_agg(n_chunks: int, with_cnt: bool):
    per_core = n_chunks // NC

    out_type = [jax.ShapeDtypeStruct((N, W), jnp.float32) for _ in range(n_chunks)]
    if with_cnt:
        out_type.append(jax.ShapeDtypeStruct((N, W), jnp.float32))

    n_in = n_chunks + (4 if with_cnt else 3)
    n_out = n_chunks + (1 if with_cnt else 0)

    scratch = [
        pltpu.VMEM((CHUNK,), jnp.int32),
        pltpu.VMEM((CHUNK,), jnp.int32),
        pltpu.VMEM((CHUNK,), jnp.int32),
        pltpu.VMEM((CHUNK,), jnp.int32),
        pltpu.VMEM((CHUNK,), jnp.int32),
        pltpu.VMEM((CHUNK, W), jnp.float32),
        pltpu.VMEM((CHUNK, W), jnp.float32),
        pltpu.VMEM_SHARED((N_ACC, W), jnp.float32),
        pltpu.SemaphoreType.DMA,
        pltpu.SemaphoreType.DMA,
    ]

    def body(*refs):
        x_refs = refs[:n_chunks]
        src_r, dst_r, z_acc = refs[n_chunks:n_chunks + 3]
        ones_h = refs[n_chunks + 3] if with_cnt else None
        o_refs = refs[n_in:n_in + n_chunks]
        cnt_o = refs[n_in + n_chunks] if with_cnt else None
        (src0, src1, dst_v, scidx0, scidx1, rows0, rows1,
         acc_sh, sem0, sem1) = refs[n_in + n_out:]

        core = lax.axis_index("c")
        sub = lax.axis_index("s")

        def _zero_acc():
            @pl.when(sub < 15)
            def _():
                pltpu.sync_copy(z_acc.at[pl.ds(sub * ZROWS, ZROWS)],
                                acc_sh.at[pl.ds(sub * ZROWS, ZROWS)])

            @pl.when(sub == 15)
            def _():
                pltpu.sync_copy(z_acc.at[pl.ds(15 * ZROWS, ZLAST)],
                                acc_sh.at[pl.ds(15 * ZROWS, ZLAST)])

        def _copy_out(o_ref, half):
            @pl.when(sub < 15)
            def _():
                pltpu.sync_copy(
                    acc_sh.at[pl.ds(sub * OROWS, OROWS)],
                    o_ref.at[pl.ds(half * NH + sub * OROWS, OROWS)])

            @pl.when(sub == 15)
            def _():
                pltpu.sync_copy(
                    acc_sh.at[pl.ds(15 * OROWS, OLAST)],
                    o_ref.at[pl.ds(half * NH + 15 * OROWS, OLAST)])

        def _transform(half, scidx_v, j):
            pltpu.sync_copy(dst_r.at[sub, j, 0], dst_v)
            base = half * NH
            for g in range(CHUNK // 16):
                d = dst_v[pl.ds(g * 16, 16)]
                dl = d - base
                ok = (dl >= 0) & (dl < NH)
                scidx_v[pl.ds(g * 16, 16)] = jnp.where(ok, dl, DUMMY)

        def run_pass(x_ref, o_ref, half):
            _zero_acc()
            plsc.subcore_barrier()

            pltpu.sync_copy(src_r.at[sub, 0, 0], src0)
            g0 = pltpu.async_copy(x_ref.at[src0], rows0, sem0)

            def step(i, carry):
                j0 = 2 * i
                j1 = j0 + 1
                _transform(half, scidx0, j0)
                pltpu.sync_copy(src_r.at[sub, j1, 0], src1)
                pltpu.make_async_copy(x_ref.at[src0], rows0, sem0).wait()
                pltpu.async_copy(x_ref.at[src1], rows1, sem1)
                pltpu.sync_copy(rows0, acc_sh.at[scidx0], add=True)
                _transform(half, scidx1, j1)
                @pl.when(i + 1 < HCPS)
                def _():
                    pltpu.sync_copy(src_r.at[sub, j0 + 2, 0], src0)
                pltpu.make_async_copy(x_ref.at[src1], rows1, sem1).wait()
                @pl.when(i + 1 < HCPS)
                def _():
                    pltpu.async_copy(x_ref.at[src0], rows0, sem0)
                pltpu.sync_copy(rows1, acc_sh.at[scidx1], add=True)
                return carry

            lax.fori_loop(0, HCPS, step, 0)
            plsc.subcore_barrier()
            _copy_out(o_ref, half)
            plsc.subcore_barrier()

        def cnt_pass(o_ref, half):
            _zero_acc()
            pltpu.sync_copy(ones_h, rows0)
            plsc.subcore_barrier()

            def step(j, carry):
                _transform(half, scidx0, j)
                pltpu.sync_copy(rows0, acc_sh.at[scidx0], add=True)
                return carry

            lax.fori_loop(0, CPS, step, 0)
            plsc.subcore_barrier()
            _copy_out(o_ref, half)
            plsc.subcore_barrier()

        for cid in range(NC):
            @pl.when(core == cid)
            def _(cid=cid):
                for k in range(per_core):
                    c = cid * per_core + k
                    for half in range(2):
                        run_pass(x_refs[c], o_refs[c], half)
                if with_cnt:
                    cnt_pass(cnt_o, cid)

    return pl.kernel(body, out_type=out_type, mesh=_MESH,
                     scratch_types=scratch)


_agg1 = _make_agg(NCH1, with_cnt=True)
_agg2 = _make_agg(NCH2, with_cnt=False)

BN = 400


def _combine1_body(*refs):
    s = refs[:NCH1]
    cnt, x, wl, wr, b = refs[NCH1:NCH1 + 5]
    o = refs[NCH1 + 5:]
    r = 1.0 / jnp.maximum(cnt[:, 0:1], 1.0)
    acc = jnp.dot(x[...], wr[...], preferred_element_type=jnp.float32)
    for c in range(NCH1):
        acc += jnp.dot(s[c][...] * r, wl[c * W:(c + 1) * W, :],
                       preferred_element_type=jnp.float32)
    h = jnp.maximum(acc + b[...], 0.0)
    for c in range(NCH2):
        o[c][...] = h[:, c * W:(c + 1) * W]


def _combine2_body(*refs):
    s = refs[:NCH2]
    cnt = refs[NCH2]
    h = refs[NCH2 + 1:NCH2 + 1 + NCH2]
    wl, wr, b, out = refs[NCH2 + 1 + NCH2:]
    r = 1.0 / jnp.maximum(cnt[:, 0:1], 1.0)
    acc = b[...] + jnp.zeros((BN, D_HID), jnp.float32)
    for c in range(NCH2):
        acc += jnp.dot(s[c][...] * r, wl[c * W:(c + 1) * W, :],
                       preferred_element_type=jnp.float32)
        acc += jnp.dot(h[c][...], wr[c * W:(c + 1) * W, :],
                       preferred_element_type=jnp.float32)
    out[...] = acc


def _row_block(d):
    return pl.BlockSpec((BN, d), lambda i: (i, 0))


def _full(shape):
    return pl.BlockSpec(shape, lambda i: tuple(0 for _ in shape))


_combine1 = pl.pallas_call(
    _combine1_body,
    grid=(N // BN,),
    in_specs=[_row_block(W)] * NCH1 + [_row_block(W), _row_block(D_IN),
              _full((D_IN, D_HID)), _full((D_IN, D_HID)), _full((1, D_HID))],
    out_specs=[_row_block(W)] * NCH2,
    out_shape=[jax.ShapeDtypeStruct((N, W), jnp.float32)] * NCH2,
)

_combine2 = pl.pallas_call(
    _combine2_body,
    grid=(N // BN,),
    in_specs=[_row_block(W)] * NCH2 + [_row_block(W)] + [_row_block(W)] * NCH2
             + [_full((D_HID, D_HID)), _full((D_HID, D_HID)), _full((1, D_HID))],
    out_specs=_row_block(D_HID),
    out_shape=jax.ShapeDtypeStruct((N, D_HID), jnp.float32),
)


def kernel(x, edge_index, W1_l, b1, W1_r, W2_l, b2, W2_r):
    src = edge_index[0].astype(jnp.int32)
    dst = edge_index[1].astype(jnp.int32)
    pad = E_PAD - E
    src_p = jnp.concatenate([src, jnp.zeros((pad,), jnp.int32)])
    src_p = src_p.reshape(NS, CPS, 1, CHUNK)
    dst_p = jnp.concatenate([dst, jnp.full((pad,), N, jnp.int32)])
    dst_p = dst_p.reshape(NS, CPS, 1, CHUNK)

    xc = [x[:, c * W:(c + 1) * W] for c in range(NCH1)]
    z_acc = jnp.zeros((N_ACC, W), jnp.float32)
    ones_h = jnp.ones((CHUNK, W), jnp.float32)

    s0, s1, cnt = _agg1(*xc, src_p, dst_p, z_acc, ones_h)
    hc = _combine1(s0, s1, cnt, x, W1_l, W1_r, b1.reshape(1, D_HID))
    t = _agg2(*hc, src_p, dst_p, z_acc)
    out = _combine2(*t, cnt, *hc, W2_l, W2_r, b2.reshape(1, D_HID))
    return out


# speedup vs baseline: 2.6658x; 1.9009x over previous
import jax
import jax.numpy as jnp
from jax import lax
from jax.experimental import pallas as pl
from jax.experimental.pallas import tpu as pltpu
from jax.experimental.pallas import tpu_sc as plsc

N = 10000
E = 160000
D_IN = 256
D_HID = 512
W = 128
NCH1 = D_IN // W
NCH2 = D_HID // W

NC = 2
NS = 16
CHUNK = 64
CPS = 160
E_PAD = NS * CPS * CHUNK
EPS = CPS * CHUNK
STG = EPS + 32
TRASH = EPS + 16

NH = N // 2
N_ACC = 5008
DUMMY = NH
ZROWS = 312
ZLAST = N_ACC - 15 * ZROWS
OROWS = 312
OLAST = NH - 15 * OROWS

_MESH = plsc.VectorSubcoreMesh(core_axis_name="c", subcore_axis_name="s")


def _bucket_body(src_r, dst_r, dummy_h, lane_h, pidx, cnts,
                 srcv, dstv, stg, cv, lanev, wsem):
    core = lax.axis_index("c")
    sub = lax.axis_index("s")
    base = core * NH

    pltpu.sync_copy(dummy_h, stg)
    pltpu.sync_copy(lane_h, lanev)

    lane = lanev[...]
    dummyv = lane * 0 + DUMMY

    def b_body(bi, carry):
        pltpu.sync_copy(src_r.at[sub, bi, 0], srcv.at[pl.ds(0, CHUNK)])
        pltpu.sync_copy(dst_r.at[sub, bi, 0], dstv.at[pl.ds(0, CHUNK)])

        def e_body(e, c):
            ptr, fill, pending = c
            sv = srcv[pl.ds(e, 16)][0]
            dl = dstv[pl.ds(e, 16)][0] - base
            keep = (dl >= 0) & (dl < NH)
            pk = (sv << 16) | (dl & 0xFFFF)
            tgt = jnp.where(keep, fill, -1)
            ins = 1 - jnp.minimum(jnp.abs(lane - tgt), 1)
            pending = pending + ins * (pk - pending)
            fill = fill + jnp.where(keep, 1, 0)

            do_flush = fill == 16

            @pl.when(do_flush)
            def _():
                stg[pl.ds(ptr, 16)] = pending

            f = jnp.where(do_flush, 1, 0)
            ptr = ptr + 16 * f
            fill = fill * (1 - f)
            pending = pending + f * (dummyv - pending)
            return ptr, fill, pending

        return lax.fori_loop(0, CHUNK, e_body, carry)

    ptr, fill, pending = lax.fori_loop(0, CPS, b_body, (0, 0, dummyv))

    @pl.when(fill > 0)
    def _():
        stg[pl.ds(ptr, 16)] = pending

    total = ptr + fill
    nch = (total + CHUNK - 1) >> 6
    nch = ((nch + 1) >> 1) << 1
    cv[...] = jnp.broadcast_to(nch >> 1, (16,)).astype(jnp.int32)
    pltpu.sync_copy(cv, cnts.at[sub, core, 0])

    def w_body(j, carry):
        pltpu.async_copy(stg.at[pl.ds(CHUNK * j, CHUNK)],
                         pidx.at[sub, core, j, 0], wsem)
        return carry

    lax.fori_loop(0, CPS, w_body, 0)

    def d_body(j, carry):
        pltpu.make_async_copy(stg.at[pl.ds(CHUNK * j, CHUNK)],
                              pidx.at[sub, core, j, 0], wsem).wait()
        return carry

    lax.fori_loop(0, CPS, d_body, 0)


_bucket = pl.kernel(
    _bucket_body,
    out_type=[
        jax.ShapeDtypeStruct((NS, NC, CPS, 1, CHUNK), jnp.int32),
        jax.ShapeDtypeStruct((NS, NC, 1, 16), jnp.int32),
    ],
    mesh=_MESH,
    scratch_types=[
        pltpu.VMEM((CHUNK + 16,), jnp.int32),
        pltpu.VMEM((CHUNK + 16,), jnp.int32),
        pltpu.VMEM((STG,), jnp.int32),
        pltpu.VMEM((16,), jnp.int32),
        pltpu.VMEM((16,), jnp.int32),
        pltpu.SemaphoreType.DMA,
    ],
)


def _make_agg(n_chunks: int, with_cnt: bool):
    per_core = n_chunks // NC

    out_type = [jax.ShapeDtypeStruct((N, W), jnp.float32) for _ in range(n_chunks)]
    if with_cnt:
        out_type.append(jax.ShapeDtypeStruct((N, W), jnp.float32))

    n_in = n_chunks + (4 if with_cnt else 3)
    n_out = n_chunks + (1 if with_cnt else 0)

    scratch = [
        pltpu.VMEM((CHUNK,), jnp.int32),
        pltpu.VMEM((CHUNK,), jnp.int32),
        pltpu.VMEM((CHUNK,), jnp.int32),
        pltpu.VMEM((CHUNK,), jnp.int32),
        pltpu.VMEM((CHUNK,), jnp.int32),
        pltpu.VMEM((CHUNK,), jnp.int32),
        pltpu.VMEM((16,), jnp.int32),
        pltpu.VMEM((CHUNK, W), jnp.float32),
        pltpu.VMEM((CHUNK, W), jnp.float32),
        pltpu.VMEM_SHARED((N_ACC, W), jnp.float32),
        pltpu.SemaphoreType.DMA,
        pltpu.SemaphoreType.DMA,
    ]

    def body(*refs):
        x_refs = refs[:n_chunks]
        pidx, cnts = refs[n_chunks:n_chunks + 2]
        z_acc = refs[n_chunks + 2]
        ones_h = refs[n_chunks + 3] if with_cnt else None
        o_refs = refs[n_in:n_in + n_chunks]
        cnt_o = refs[n_in + n_chunks] if with_cnt else None
        (pk0, pk1, src0, src1, sci0, sci1, cv, rows0, rows1,
         acc_sh, sem0, sem1) = refs[n_in + n_out:]

        core = lax.axis_index("c")
        sub = lax.axis_index("s")

        def _zero_acc():
            @pl.when(sub < 15)
            def _():
                pltpu.sync_copy(z_acc.at[pl.ds(sub * ZROWS, ZROWS)],
                                acc_sh.at[pl.ds(sub * ZROWS, ZROWS)])

            @pl.when(sub == 15)
            def _():
                pltpu.sync_copy(z_acc.at[pl.ds(15 * ZROWS, ZLAST)],
                                acc_sh.at[pl.ds(15 * ZROWS, ZLAST)])

        def _copy_out(o_ref, half):
            @pl.when(sub < 15)
            def _():
                pltpu.sync_copy(
                    acc_sh.at[pl.ds(sub * OROWS, OROWS)],
                    o_ref.at[pl.ds(half * NH + sub * OROWS, OROWS)])

            @pl.when(sub == 15)
            def _():
                pltpu.sync_copy(
                    acc_sh.at[pl.ds(15 * OROWS, OLAST)],
                    o_ref.at[pl.ds(half * NH + 15 * OROWS, OLAST)])

        def _load_unpack(half, j, pk_v, src_v, sci_v, need_src=True):
            pltpu.sync_copy(pidx.at[sub, half, j, 0], pk_v)
            for g in range(CHUNK // 16):
                v = pk_v[pl.ds(g * 16, 16)]
                if need_src:
                    src_v[pl.ds(g * 16, 16)] = v >> 16
                sci_v[pl.ds(g * 16, 16)] = v & 0xFFFF

        def _npair(half):
            pltpu.sync_copy(cnts.at[sub, half, 0], cv)
            return cv[...][0]

        def run_pass(x_ref, o_ref, half):
            _zero_acc()
            plsc.subcore_barrier()
            npair = _npair(half)

            @pl.when(npair > 0)
            def _():
                _load_unpack(half, 0, pk0, src0, sci0)
                pltpu.async_copy(x_ref.at[src0], rows0, sem0)

                def step(i, carry):
                    j1 = 2 * i + 1
                    _load_unpack(half, j1, pk1, src1, sci1)
                    pltpu.make_async_copy(x_ref.at[src0], rows0, sem0).wait()
                    pltpu.async_copy(x_ref.at[src1], rows1, sem1)
                    pltpu.sync_copy(rows0, acc_sh.at[sci0], add=True)

                    @pl.when(i + 1 < npair)
                    def _():
                        _load_unpack(half, j1 + 1, pk0, src0, sci0)
                    pltpu.make_async_copy(x_ref.at[src1], rows1, sem1).wait()

                    @pl.when(i + 1 < npair)
                    def _():
                        pltpu.async_copy(x_ref.at[src0], rows0, sem0)
                    pltpu.sync_copy(rows1, acc_sh.at[sci1], add=True)
                    return carry

                lax.fori_loop(0, npair, step, 0)

            plsc.subcore_barrier()
            _copy_out(o_ref, half)
            plsc.subcore_barrier()

        def cnt_pass(o_ref, half):
            _zero_acc()
            pltpu.sync_copy(ones_h, rows0)
            plsc.subcore_barrier()
            nch = 2 * _npair(half)

            def step(j, carry):
                _load_unpack(half, j, pk0, src0, sci0, need_src=False)
                pltpu.sync_copy(rows0, acc_sh.at[sci0], add=True)
                return carry

            lax.fori_loop(0, nch, step, 0)
            plsc.subcore_barrier()
            _copy_out(o_ref, half)
            plsc.subcore_barrier()

        for cid in range(NC):
            @pl.when(core == cid)
            def _(cid=cid):
                for k in range(per_core):
                    c = cid * per_core + k
                    for half in range(2):
                        run_pass(x_refs[c], o_refs[c], half)
                if with_cnt:
                    cnt_pass(cnt_o, cid)

    return pl.kernel(body, out_type=out_type, mesh=_MESH,
                     scratch_types=scratch)


_agg1 = _make_agg(NCH1, with_cnt=True)
_agg2 = _make_agg(NCH2, with_cnt=False)

BN = 400


def _combine1_body(*refs):
    s = refs[:NCH1]
    cnt, x, wl, wr, b = refs[NCH1:NCH1 + 5]
    o = refs[NCH1 + 5:]
    r = 1.0 / jnp.maximum(cnt[:, 0:1], 1.0)
    acc = jnp.dot(x[...], wr[...], preferred_element_type=jnp.float32)
    for c in range(NCH1):
        acc += jnp.dot(s[c][...] * r, wl[c * W:(c + 1) * W, :],
                       preferred_element_type=jnp.float32)
    h = jnp.maximum(acc + b[...], 0.0)
    for c in range(NCH2):
        o[c][...] = h[:, c * W:(c + 1) * W]


def _combine2_body(*refs):
    s = refs[:NCH2]
    cnt = refs[NCH2]
    h = refs[NCH2 + 1:NCH2 + 1 + NCH2]
    wl, wr, b, out = refs[NCH2 + 1 + NCH2:]
    r = 1.0 / jnp.maximum(cnt[:, 0:1], 1.0)
    acc = b[...] + jnp.zeros((BN, D_HID), jnp.float32)
    for c in range(NCH2):
        acc += jnp.dot(s[c][...] * r, wl[c * W:(c + 1) * W, :],
                       preferred_element_type=jnp.float32)
        acc += jnp.dot(h[c][...], wr[c * W:(c + 1) * W, :],
                       preferred_element_type=jnp.float32)
    out[...] = acc


def _row_block(d):
    return pl.BlockSpec((BN, d), lambda i: (i, 0))


def _full(shape):
    return pl.BlockSpec(shape, lambda i: tuple(0 for _ in shape))


_combine1 = pl.pallas_call(
    _combine1_body,
    grid=(N // BN,),
    in_specs=[_row_block(W)] * NCH1 + [_row_block(W), _row_block(D_IN),
              _full((D_IN, D_HID)), _full((D_IN, D_HID)), _full((1, D_HID))],
    out_specs=[_row_block(W)] * NCH2,
    out_shape=[jax.ShapeDtypeStruct((N, W), jnp.float32)] * NCH2,
)

_combine2 = pl.pallas_call(
    _combine2_body,
    grid=(N // BN,),
    in_specs=[_row_block(W)] * NCH2 + [_row_block(W)] + [_row_block(W)] * NCH2
             + [_full((D_HID, D_HID)), _full((D_HID, D_HID)), _full((1, D_HID))],
    out_specs=_row_block(D_HID),
    out_shape=jax.ShapeDtypeStruct((N, D_HID), jnp.float32),
)


def kernel(x, edge_index, W1_l, b1, W1_r, W2_l, b2, W2_r):
    src = edge_index[0].astype(jnp.int32)
    dst = edge_index[1].astype(jnp.int32)
    pad = E_PAD - E
    src_p = jnp.concatenate([src, jnp.zeros((pad,), jnp.int32)])
    src_p = src_p.reshape(NS, CPS, 1, CHUNK)
    dst_p = jnp.concatenate([dst, jnp.full((pad,), N, jnp.int32)])
    dst_p = dst_p.reshape(NS, CPS, 1, CHUNK)

    xc = [x[:, c * W:(c + 1) * W] for c in range(NCH1)]
    z_acc = jnp.zeros((N_ACC, W), jnp.float32)
    ones_h = jnp.ones((CHUNK, W), jnp.float32)
    dummy_h = jnp.full((STG,), DUMMY, jnp.int32)

    lane_h = jnp.arange(16, dtype=jnp.int32)
    pidx, cnts = _bucket(src_p, dst_p, dummy_h, lane_h)
    s0, s1, cnt = _agg1(*xc, pidx, cnts, z_acc, ones_h)
    hc = _combine1(s0, s1, cnt, x, W1_l, W1_r, b1.reshape(1, D_HID))
    t = _agg2(*hc, pidx, cnts, z_acc)
    out = _combine2(*t, cnt, *hc, W2_l, W2_r, b2.reshape(1, D_HID))
    return out


# speedup vs baseline: 2.9482x; 1.1059x over previous
import jax
import jax.numpy as jnp
from jax import lax
from jax.experimental import pallas as pl
from jax.experimental.pallas import tpu as pltpu
from jax.experimental.pallas import tpu_sc as plsc

N = 10000
E = 160000
D_IN = 256
D_HID = 512
W = 128
NCH1 = D_IN // W
NCH2 = D_HID // W

NC = 2
NS = 16
CHUNK = 64
CPS = 160
E_PAD = NS * CPS * CHUNK
EPS = CPS * CHUNK
STG = EPS + 32
BLK = 2048

NH = N // 2
N_ACC = 5008
DUMMY = NH
ZROWS = 312
ZLAST = N_ACC - 15 * ZROWS
OROWS = 312
OLAST = NH - 15 * OROWS

_MESH = plsc.VectorSubcoreMesh(core_axis_name="c", subcore_axis_name="s")


def _bucket_body(src_r, dst_r, dummy_h, lane_h, pidx, cnts,
                 srcv, dstv, stg, cv, lanev, wsem):
    core = lax.axis_index("c")
    sub = lax.axis_index("s")
    base = core * NH

    pltpu.sync_copy(dummy_h, stg)
    pltpu.sync_copy(lane_h, lanev)
    lane = lanev[...]
    dummyv = lane * 0 + DUMMY

    def b_body(bi, carry):
        pltpu.sync_copy(src_r.at[sub, 0, pl.ds(bi * BLK, BLK)],
                        srcv.at[pl.ds(0, BLK)])
        pltpu.sync_copy(dst_r.at[sub, 0, pl.ds(bi * BLK, BLK)],
                        dstv.at[pl.ds(0, BLK)])

        def e_body(e, c):
            ptr, fill, pending = c
            sv = srcv[pl.ds(e, 16)][0]
            dl = dstv[pl.ds(e, 16)][0] - base
            keep = (dl >= 0) & (dl < NH)
            pk = (sv << 16) | (dl & 0xFFFF)
            tgt = jnp.where(keep, fill, -1)
            ins = 1 - jnp.minimum(jnp.abs(lane - tgt), 1)
            pending = pending + ins * (pk - pending)
            fill = fill + jnp.where(keep, 1, 0)

            do_flush = fill == 16

            @pl.when(do_flush)
            def _():
                stg[pl.ds(ptr, 16)] = pending

            f = jnp.where(do_flush, 1, 0)
            ptr = ptr + 16 * f
            fill = fill * (1 - f)
            pending = pending + f * (dummyv - pending)
            return ptr, fill, pending

        return lax.fori_loop(0, BLK, e_body, carry)

    ptr, fill, pending = lax.fori_loop(0, EPS // BLK, b_body, (0, 0, dummyv))

    @pl.when(fill > 0)
    def _():
        stg[pl.ds(ptr, 16)] = pending

    total = ptr + fill
    nch = (total + CHUNK - 1) >> 6
    nch = ((nch + 1) >> 1) << 1
    cv[...] = jnp.broadcast_to(nch >> 1, (16,)).astype(jnp.int32)
    pltpu.sync_copy(cv, cnts.at[sub, core, 0])

    def w_body(j, carry):
        pltpu.async_copy(stg.at[pl.ds(CHUNK * j, CHUNK)],
                         pidx.at[sub, core, j, 0], wsem)
        return carry

    lax.fori_loop(0, CPS, w_body, 0)

    def d_body(j, carry):
        pltpu.make_async_copy(stg.at[pl.ds(CHUNK * j, CHUNK)],
                              pidx.at[sub, core, j, 0], wsem).wait()
        return carry

    lax.fori_loop(0, CPS, d_body, 0)


_bucket = pl.kernel(
    _bucket_body,
    out_type=[
        jax.ShapeDtypeStruct((NS, NC, CPS, 1, CHUNK), jnp.int32),
        jax.ShapeDtypeStruct((NS, NC, 1, 16), jnp.int32),
    ],
    mesh=_MESH,
    scratch_types=[
        pltpu.VMEM((BLK + 16,), jnp.int32),
        pltpu.VMEM((BLK + 16,), jnp.int32),
        pltpu.VMEM((STG,), jnp.int32),
        pltpu.VMEM((16,), jnp.int32),
        pltpu.VMEM((16,), jnp.int32),
        pltpu.SemaphoreType.DMA,
    ],
)


def _make_agg(n_chunks: int, with_cnt: bool):
    per_core = n_chunks // NC

    out_type = [jax.ShapeDtypeStruct((N, W), jnp.float32) for _ in range(n_chunks)]
    if with_cnt:
        out_type.append(jax.ShapeDtypeStruct((N, W), jnp.float32))

    n_in = n_chunks + (4 if with_cnt else 3)
    n_out = n_chunks + (1 if with_cnt else 0)

    scratch = [
        pltpu.VMEM((CHUNK,), jnp.int32),
        pltpu.VMEM((CHUNK,), jnp.int32),
        pltpu.VMEM((CHUNK,), jnp.int32),
        pltpu.VMEM((CHUNK,), jnp.int32),
        pltpu.VMEM((CHUNK,), jnp.int32),
        pltpu.VMEM((CHUNK,), jnp.int32),
        pltpu.VMEM((16,), jnp.int32),
        pltpu.VMEM((CHUNK, W), jnp.float32),
        pltpu.VMEM((CHUNK, W), jnp.float32),
        pltpu.VMEM_SHARED((N_ACC, W), jnp.float32),
        pltpu.SemaphoreType.DMA,
        pltpu.SemaphoreType.DMA,
    ]

    def body(*refs):
        x_refs = refs[:n_chunks]
        pidx, cnts = refs[n_chunks:n_chunks + 2]
        z_acc = refs[n_chunks + 2]
        ones_h = refs[n_chunks + 3] if with_cnt else None
        o_refs = refs[n_in:n_in + n_chunks]
        cnt_o = refs[n_in + n_chunks] if with_cnt else None
        (pk0, pk1, src0, src1, sci0, sci1, cv, rows0, rows1,
         acc_sh, sem0, sem1) = refs[n_in + n_out:]

        core = lax.axis_index("c")
        sub = lax.axis_index("s")

        def _zero_acc():
            @pl.when(sub < 15)
            def _():
                pltpu.sync_copy(z_acc.at[pl.ds(sub * ZROWS, ZROWS)],
                                acc_sh.at[pl.ds(sub * ZROWS, ZROWS)])

            @pl.when(sub == 15)
            def _():
                pltpu.sync_copy(z_acc.at[pl.ds(15 * ZROWS, ZLAST)],
                                acc_sh.at[pl.ds(15 * ZROWS, ZLAST)])

        def _copy_out(o_ref, half):
            @pl.when(sub < 15)
            def _():
                pltpu.sync_copy(
                    acc_sh.at[pl.ds(sub * OROWS, OROWS)],
                    o_ref.at[pl.ds(half * NH + sub * OROWS, OROWS)])

            @pl.when(sub == 15)
            def _():
                pltpu.sync_copy(
                    acc_sh.at[pl.ds(15 * OROWS, OLAST)],
                    o_ref.at[pl.ds(half * NH + 15 * OROWS, OLAST)])

        def _load_unpack(half, j, pk_v, src_v, sci_v, need_src=True):
            pltpu.sync_copy(pidx.at[sub, half, j, 0], pk_v)
            for g in range(CHUNK // 16):
                v = pk_v[pl.ds(g * 16, 16)]
                if need_src:
                    src_v[pl.ds(g * 16, 16)] = v >> 16
                sci_v[pl.ds(g * 16, 16)] = v & 0xFFFF

        def _npair(half):
            pltpu.sync_copy(cnts.at[sub, half, 0], cv)
            return cv[...][0]

        def run_pass(x_ref, o_ref, half):
            _zero_acc()
            plsc.subcore_barrier()
            npair = _npair(half)

            @pl.when(npair > 0)
            def _():
                _load_unpack(half, 0, pk0, src0, sci0)
                pltpu.async_copy(x_ref.at[src0], rows0, sem0)

                def step(i, carry):
                    j1 = 2 * i + 1
                    _load_unpack(half, j1, pk1, src1, sci1)
                    pltpu.make_async_copy(x_ref.at[src0], rows0, sem0).wait()
                    pltpu.async_copy(x_ref.at[src1], rows1, sem1)
                    pltpu.sync_copy(rows0, acc_sh.at[sci0], add=True)

                    @pl.when(i + 1 < npair)
                    def _():
                        _load_unpack(half, j1 + 1, pk0, src0, sci0)
                    pltpu.make_async_copy(x_ref.at[src1], rows1, sem1).wait()

                    @pl.when(i + 1 < npair)
                    def _():
                        pltpu.async_copy(x_ref.at[src0], rows0, sem0)
                    pltpu.sync_copy(rows1, acc_sh.at[sci1], add=True)
                    return carry

                lax.fori_loop(0, npair, step, 0)

            plsc.subcore_barrier()
            _copy_out(o_ref, half)
            plsc.subcore_barrier()

        def cnt_pass(o_ref, half):
            _zero_acc()
            pltpu.sync_copy(ones_h, rows0)
            plsc.subcore_barrier()
            nch = 2 * _npair(half)

            def step(j, carry):
                _load_unpack(half, j, pk0, src0, sci0, need_src=False)
                pltpu.sync_copy(rows0, acc_sh.at[sci0], add=True)
                return carry

            lax.fori_loop(0, nch, step, 0)
            plsc.subcore_barrier()
            _copy_out(o_ref, half)
            plsc.subcore_barrier()

        for cid in range(NC):
            @pl.when(core == cid)
            def _(cid=cid):
                for k in range(per_core):
                    c = cid * per_core + k
                    for half in range(2):
                        run_pass(x_refs[c], o_refs[c], half)
                if with_cnt:
                    cnt_pass(cnt_o, cid)

    return pl.kernel(body, out_type=out_type, mesh=_MESH,
                     scratch_types=scratch)


_agg1 = _make_agg(NCH1, with_cnt=True)
_agg2 = _make_agg(NCH2, with_cnt=False)

BN = 400


def _combine1_body(*refs):
    s = refs[:NCH1]
    cnt, x, wl, wr, b = refs[NCH1:NCH1 + 5]
    o = refs[NCH1 + 5:]
    r = 1.0 / jnp.maximum(cnt[:, 0:1], 1.0)
    acc = jnp.dot(x[...], wr[...], preferred_element_type=jnp.float32)
    for c in range(NCH1):
        acc += jnp.dot(s[c][...] * r, wl[c * W:(c + 1) * W, :],
                       preferred_element_type=jnp.float32)
    h = jnp.maximum(acc + b[...], 0.0)
    for c in range(NCH2):
        o[c][...] = h[:, c * W:(c + 1) * W]


def _combine2_body(*refs):
    s = refs[:NCH2]
    cnt = refs[NCH2]
    h = refs[NCH2 + 1:NCH2 + 1 + NCH2]
    wl, wr, b, out = refs[NCH2 + 1 + NCH2:]
    r = 1.0 / jnp.maximum(cnt[:, 0:1], 1.0)
    acc = b[...] + jnp.zeros((BN, D_HID), jnp.float32)
    for c in range(NCH2):
        acc += jnp.dot(s[c][...] * r, wl[c * W:(c + 1) * W, :],
                       preferred_element_type=jnp.float32)
        acc += jnp.dot(h[c][...], wr[c * W:(c + 1) * W, :],
                       preferred_element_type=jnp.float32)
    out[...] = acc


def _row_block(d):
    return pl.BlockSpec((BN, d), lambda i: (i, 0))


def _full(shape):
    return pl.BlockSpec(shape, lambda i: tuple(0 for _ in shape))


_combine1 = pl.pallas_call(
    _combine1_body,
    grid=(N // BN,),
    in_specs=[_row_block(W)] * NCH1 + [_row_block(W), _row_block(D_IN),
              _full((D_IN, D_HID)), _full((D_IN, D_HID)), _full((1, D_HID))],
    out_specs=[_row_block(W)] * NCH2,
    out_shape=[jax.ShapeDtypeStruct((N, W), jnp.float32)] * NCH2,
)

_combine2 = pl.pallas_call(
    _combine2_body,
    grid=(N // BN,),
    in_specs=[_row_block(W)] * NCH2 + [_row_block(W)] + [_row_block(W)] * NCH2
             + [_full((D_HID, D_HID)), _full((D_HID, D_HID)), _full((1, D_HID))],
    out_specs=_row_block(D_HID),
    out_shape=jax.ShapeDtypeStruct((N, D_HID), jnp.float32),
)


def kernel(x, edge_index, W1_l, b1, W1_r, W2_l, b2, W2_r):
    src = edge_index[0].astype(jnp.int32)
    dst = edge_index[1].astype(jnp.int32)
    pad = E_PAD - E
    src_p = jnp.concatenate([src, jnp.zeros((pad,), jnp.int32)])
    src_p = src_p.reshape(NS, CPS, 1, CHUNK)
    dst_p = jnp.concatenate([dst, jnp.full((pad,), N, jnp.int32)])
    dst_p = dst_p.reshape(NS, CPS, 1, CHUNK)

    xc = [x[:, c * W:(c + 1) * W] for c in range(NCH1)]
    z_acc = jnp.zeros((N_ACC, W), jnp.float32)
    ones_h = jnp.ones((CHUNK, W), jnp.float32)
    dummy_h = jnp.full((STG,), DUMMY, jnp.int32)

    lane_h = jnp.arange(16, dtype=jnp.int32)
    src_f = src_p.reshape(NS, 1, EPS)
    dst_f = dst_p.reshape(NS, 1, EPS)
    pidx, cnts = _bucket(src_f, dst_f, dummy_h, lane_h)
    s0, s1, cnt = _agg1(*xc, pidx, cnts, z_acc, ones_h)
    hc = _combine1(s0, s1, cnt, x, W1_l, W1_r, b1.reshape(1, D_HID))
    t = _agg2(*hc, pidx, cnts, z_acc)
    out = _combine2(*t, cnt, *hc, W2_l, W2_r, b2.reshape(1, D_HID))
    return out


# speedup vs baseline: 3.0306x; 1.0280x over previous
import jax
import jax.numpy as jnp
from jax import lax
from jax.experimental import pallas as pl
from jax.experimental.pallas import tpu as pltpu
from jax.experimental.pallas import tpu_sc as plsc

N = 10000
E = 160000
D_IN = 256
D_HID = 512
W = 128
NCH1 = D_IN // W
NCH2 = D_HID // W

NC = 2
NS = 16
CHUNK = 64
CPS = 160
E_PAD = NS * CPS * CHUNK
EPS = CPS * CHUNK
STG = EPS + 32
BLK = 2048
PKB = 16

NH = N // 2
N_ACC = 5008
DUMMY = NH
ZROWS = 312
ZLAST = N_ACC - 15 * ZROWS
OROWS = 312
OLAST = NH - 15 * OROWS

_MESH = plsc.VectorSubcoreMesh(core_axis_name="c", subcore_axis_name="s")


def _bucket_body(src_r, dst_r, dummy_h, lane_h, pidx, cnts,
                 srcv, dstv, stg, cv, lanev, wsem):
    core = lax.axis_index("c")
    sub = lax.axis_index("s")
    base = core * NH

    pltpu.sync_copy(dummy_h, stg)
    pltpu.sync_copy(lane_h, lanev)
    lane = lanev[...]
    dummyv = lane * 0 + DUMMY

    def b_body(bi, carry):
        pltpu.sync_copy(src_r.at[sub, 0, pl.ds(bi * BLK, BLK)],
                        srcv.at[pl.ds(0, BLK)])
        pltpu.sync_copy(dst_r.at[sub, 0, pl.ds(bi * BLK, BLK)],
                        dstv.at[pl.ds(0, BLK)])

        def e_body(e, c):
            ptr, fill, pending = c
            sv = srcv[pl.ds(e, 16)][0]
            dl = dstv[pl.ds(e, 16)][0] - base
            keep = (dl >= 0) & (dl < NH)
            pk = (sv << 16) | (dl & 0xFFFF)
            tgt = jnp.where(keep, fill, -1)
            ins = 1 - jnp.minimum(jnp.abs(lane - tgt), 1)
            pending = pending + ins * (pk - pending)
            fill = fill + jnp.where(keep, 1, 0)

            do_flush = fill == 16

            @pl.when(do_flush)
            def _():
                stg[pl.ds(ptr, 16)] = pending

            f = jnp.where(do_flush, 1, 0)
            ptr = ptr + 16 * f
            fill = fill * (1 - f)
            pending = pending + f * (dummyv - pending)
            return ptr, fill, pending

        return lax.fori_loop(0, BLK, e_body, carry)

    ptr, fill, pending = lax.fori_loop(0, EPS // BLK, b_body, (0, 0, dummyv))

    @pl.when(fill > 0)
    def _():
        stg[pl.ds(ptr, 16)] = pending

    total = ptr + fill
    nch = (total + CHUNK - 1) >> 6
    nch = ((nch + 1) >> 1) << 1
    cv[...] = jnp.broadcast_to(nch >> 1, (16,)).astype(jnp.int32)
    pltpu.sync_copy(cv, cnts.at[sub, core, 0])

    def w_body(j, carry):
        pltpu.async_copy(stg.at[pl.ds(CHUNK * j, CHUNK)],
                         pidx.at[sub, core, j, 0], wsem)
        return carry

    lax.fori_loop(0, CPS, w_body, 0)

    def d_body(j, carry):
        pltpu.make_async_copy(stg.at[pl.ds(CHUNK * j, CHUNK)],
                              pidx.at[sub, core, j, 0], wsem).wait()
        return carry

    lax.fori_loop(0, CPS, d_body, 0)


_bucket = pl.kernel(
    _bucket_body,
    out_type=[
        jax.ShapeDtypeStruct((NS, NC, CPS, 1, CHUNK), jnp.int32),
        jax.ShapeDtypeStruct((NS, NC, 1, 16), jnp.int32),
    ],
    mesh=_MESH,
    scratch_types=[
        pltpu.VMEM((BLK + 16,), jnp.int32),
        pltpu.VMEM((BLK + 16,), jnp.int32),
        pltpu.VMEM((STG,), jnp.int32),
        pltpu.VMEM((16,), jnp.int32),
        pltpu.VMEM((16,), jnp.int32),
        pltpu.SemaphoreType.DMA,
    ],
)


def _make_agg(n_chunks: int, with_cnt: bool):
    per_core = n_chunks // NC

    out_type = [jax.ShapeDtypeStruct((N, W), jnp.float32) for _ in range(n_chunks)]
    if with_cnt:
        out_type.append(jax.ShapeDtypeStruct((N, W), jnp.float32))

    n_in = n_chunks + (4 if with_cnt else 3)
    n_out = n_chunks + (1 if with_cnt else 0)

    scratch = [
        pltpu.VMEM((PKB * CHUNK,), jnp.int32),
        pltpu.VMEM((CHUNK,), jnp.int32),
        pltpu.VMEM((CHUNK,), jnp.int32),
        pltpu.VMEM((CHUNK,), jnp.int32),
        pltpu.VMEM((CHUNK,), jnp.int32),
        pltpu.VMEM((16,), jnp.int32),
        pltpu.VMEM((CHUNK, W), jnp.float32),
        pltpu.VMEM((CHUNK, W), jnp.float32),
        pltpu.VMEM_SHARED((N_ACC, W), jnp.float32),
        pltpu.SemaphoreType.DMA,
        pltpu.SemaphoreType.DMA,
    ]

    def body(*refs):
        x_refs = refs[:n_chunks]
        pidx_f, cnts = refs[n_chunks:n_chunks + 2]
        z_acc = refs[n_chunks + 2]
        ones_h = refs[n_chunks + 3] if with_cnt else None
        o_refs = refs[n_in:n_in + n_chunks]
        cnt_o = refs[n_in + n_chunks] if with_cnt else None
        (pkblk, src0, src1, sci0, sci1, cv, rows0, rows1,
         acc_sh, sem0, sem1) = refs[n_in + n_out:]

        core = lax.axis_index("c")
        sub = lax.axis_index("s")

        def _zero_acc():
            @pl.when(sub < 15)
            def _():
                pltpu.sync_copy(z_acc.at[pl.ds(sub * ZROWS, ZROWS)],
                                acc_sh.at[pl.ds(sub * ZROWS, ZROWS)])

            @pl.when(sub == 15)
            def _():
                pltpu.sync_copy(z_acc.at[pl.ds(15 * ZROWS, ZLAST)],
                                acc_sh.at[pl.ds(15 * ZROWS, ZLAST)])

        def _copy_out(o_ref, half):
            @pl.when(sub < 15)
            def _():
                pltpu.sync_copy(
                    acc_sh.at[pl.ds(sub * OROWS, OROWS)],
                    o_ref.at[pl.ds(half * NH + sub * OROWS, OROWS)])

            @pl.when(sub == 15)
            def _():
                pltpu.sync_copy(
                    acc_sh.at[pl.ds(15 * OROWS, OLAST)],
                    o_ref.at[pl.ds(half * NH + 15 * OROWS, OLAST)])

        def _load_unpack(half, j, src_v, sci_v, need_src=True):
            @pl.when((j & (PKB - 1)) == 0)
            def _():
                pltpu.sync_copy(
                    pidx_f.at[sub, half, 0,
                              pl.ds((j >> 4) * (PKB * CHUNK), PKB * CHUNK)],
                    pkblk)

            loc = (j & (PKB - 1)) * CHUNK
            for g in range(CHUNK // 16):
                v = pkblk[pl.ds(loc + g * 16, 16)]
                if need_src:
                    src_v[pl.ds(g * 16, 16)] = v >> 16
                sci_v[pl.ds(g * 16, 16)] = v & 0xFFFF

        def _npair(half):
            pltpu.sync_copy(cnts.at[sub, half, 0], cv)
            return cv[...][0]

        def run_pass(x_ref, o_ref, half):
            _zero_acc()
            plsc.subcore_barrier()
            npair = _npair(half)

            @pl.when(npair > 0)
            def _():
                _load_unpack(half, 0, src0, sci0)
                pltpu.async_copy(x_ref.at[src0], rows0, sem0)

                def step(i, carry):
                    j1 = 2 * i + 1
                    _load_unpack(half, j1, src1, sci1)
                    pltpu.make_async_copy(x_ref.at[src0], rows0, sem0).wait()
                    pltpu.async_copy(x_ref.at[src1], rows1, sem1)
                    pltpu.sync_copy(rows0, acc_sh.at[sci0], add=True)

                    @pl.when(i + 1 < npair)
                    def _():
                        _load_unpack(half, j1 + 1, src0, sci0)
                    pltpu.make_async_copy(x_ref.at[src1], rows1, sem1).wait()

                    @pl.when(i + 1 < npair)
                    def _():
                        pltpu.async_copy(x_ref.at[src0], rows0, sem0)
                    pltpu.sync_copy(rows1, acc_sh.at[sci1], add=True)
                    return carry

                lax.fori_loop(0, npair, step, 0)

            plsc.subcore_barrier()
            _copy_out(o_ref, half)
            plsc.subcore_barrier()

        def cnt_pass(o_ref, half):
            _zero_acc()
            pltpu.sync_copy(ones_h, rows0)
            plsc.subcore_barrier()
            nch = 2 * _npair(half)

            def step(j, carry):
                _load_unpack(half, j, src0, sci0, need_src=False)
                pltpu.sync_copy(rows0, acc_sh.at[sci0], add=True)
                return carry

            lax.fori_loop(0, nch, step, 0)
            plsc.subcore_barrier()
            _copy_out(o_ref, half)
            plsc.subcore_barrier()

        for cid in range(NC):
            @pl.when(core == cid)
            def _(cid=cid):
                for k in range(per_core):
                    c = cid * per_core + k
                    for half in range(2):
                        run_pass(x_refs[c], o_refs[c], half)
                if with_cnt:
                    cnt_pass(cnt_o, cid)

    return pl.kernel(body, out_type=out_type, mesh=_MESH,
                     scratch_types=scratch)


_agg1 = _make_agg(NCH1, with_cnt=True)
_agg2 = _make_agg(NCH2, with_cnt=False)

BN = 400


def _combine1_body(*refs):
    s = refs[:NCH1]
    cnt, x, wl, wr, b = refs[NCH1:NCH1 + 5]
    o = refs[NCH1 + 5:]
    r = 1.0 / jnp.maximum(cnt[:, 0:1], 1.0)
    acc = jnp.dot(x[...], wr[...], preferred_element_type=jnp.float32)
    for c in range(NCH1):
        acc += jnp.dot(s[c][...] * r, wl[c * W:(c + 1) * W, :],
                       preferred_element_type=jnp.float32)
    h = jnp.maximum(acc + b[...], 0.0)
    for c in range(NCH2):
        o[c][...] = h[:, c * W:(c + 1) * W]


def _combine2_body(*refs):
    s = refs[:NCH2]
    cnt = refs[NCH2]
    h = refs[NCH2 + 1:NCH2 + 1 + NCH2]
    wl, wr, b, out = refs[NCH2 + 1 + NCH2:]
    r = 1.0 / jnp.maximum(cnt[:, 0:1], 1.0)
    acc = b[...] + jnp.zeros((BN, D_HID), jnp.float32)
    for c in range(NCH2):
        acc += jnp.dot(s[c][...] * r, wl[c * W:(c + 1) * W, :],
                       preferred_element_type=jnp.float32)
        acc += jnp.dot(h[c][...], wr[c * W:(c + 1) * W, :],
                       preferred_element_type=jnp.float32)
    out[...] = acc


def _row_block(d):
    return pl.BlockSpec((BN, d), lambda i: (i, 0))


def _full(shape):
    return pl.BlockSpec(shape, lambda i: tuple(0 for _ in shape))


_combine1 = pl.pallas_call(
    _combine1_body,
    grid=(N // BN,),
    in_specs=[_row_block(W)] * NCH1 + [_row_block(W), _row_block(D_IN),
              _full((D_IN, D_HID)), _full((D_IN, D_HID)), _full((1, D_HID))],
    out_specs=[_row_block(W)] * NCH2,
    out_shape=[jax.ShapeDtypeStruct((N, W), jnp.float32)] * NCH2,
)

_combine2 = pl.pallas_call(
    _combine2_body,
    grid=(N // BN,),
    in_specs=[_row_block(W)] * NCH2 + [_row_block(W)] + [_row_block(W)] * NCH2
             + [_full((D_HID, D_HID)), _full((D_HID, D_HID)), _full((1, D_HID))],
    out_specs=_row_block(D_HID),
    out_shape=jax.ShapeDtypeStruct((N, D_HID), jnp.float32),
)


def kernel(x, edge_index, W1_l, b1, W1_r, W2_l, b2, W2_r):
    src = edge_index[0].astype(jnp.int32)
    dst = edge_index[1].astype(jnp.int32)
    pad = E_PAD - E
    src_p = jnp.concatenate([src, jnp.zeros((pad,), jnp.int32)])
    src_p = src_p.reshape(NS, CPS, 1, CHUNK)
    dst_p = jnp.concatenate([dst, jnp.full((pad,), N, jnp.int32)])
    dst_p = dst_p.reshape(NS, CPS, 1, CHUNK)

    xc = [x[:, c * W:(c + 1) * W] for c in range(NCH1)]
    z_acc = jnp.zeros((N_ACC, W), jnp.float32)
    ones_h = jnp.ones((CHUNK, W), jnp.float32)
    dummy_h = jnp.full((STG,), DUMMY, jnp.int32)

    lane_h = jnp.arange(16, dtype=jnp.int32)
    src_f = src_p.reshape(NS, 1, EPS)
    dst_f = dst_p.reshape(NS, 1, EPS)
    pidx, cnts = _bucket(src_f, dst_f, dummy_h, lane_h)
    pidx_f = pidx.reshape(NS, NC, 1, CPS * CHUNK)
    s0, s1, cnt = _agg1(*xc, pidx_f, cnts, z_acc, ones_h)
    hc = _combine1(s0, s1, cnt, x, W1_l, W1_r, b1.reshape(1, D_HID))
    t = _agg2(*hc, pidx_f, cnts, z_acc)
    out = _combine2(*t, cnt, *hc, W2_l, W2_r, b2.reshape(1, D_HID))
    return out
